# baseline (speedup 1.0000x reference)
import jax
import jax.numpy as jnp
import numpy as np
from jax.experimental import pallas as pl
from jax.experimental.pallas import tpu as pltpu

K = 100
N = 50
K_N = K * N
E = K * N * N
D = 128
TDIM = 16
NN = N * N
G = 2
GN = G * N
GNN = G * NN


def _edge_kernel(znode_ref, zedge_ref, treat_ref,
                 wea_ref, web_ref, wec_ref, wed_ref,
                 w1a_ref, w1b_ref, be_ref, wv_ref,
                 p_ref, rt_ref, c_ref,
                 out_ref, msg_ref):
    f32 = jnp.float32
    nb = znode_ref[0]
    tr = treat_ref[0]
    a2 = (jnp.dot(nb, wea_ref[...], preferred_element_type=f32)
          + jnp.dot(tr, web_ref[...], preferred_element_type=f32))
    b2 = (jnp.dot(nb, wec_ref[...], preferred_element_type=f32)
          + jnp.dot(tr, wed_ref[...], preferred_element_type=f32))
    x2 = jnp.tanh(jnp.dot(nb, w1a_ref[...], preferred_element_type=f32)
                  + jnp.dot(tr, w1b_ref[...], preferred_element_type=f32))

    for g in range(G):
        edges = zedge_ref[0, g * NN:(g + 1) * NN]
        a = a2[g * N:(g + 1) * N]
        b = b2[g * N:(g + 1) * N]
        x = x2[g * N:(g + 1) * N]
        ab = jnp.concatenate([a, b], axis=0)
        u = jnp.dot(p_ref[...], ab, preferred_element_type=f32)
        out_ref[0, g * NN:(g + 1) * NN] = jnp.tanh(u + be_ref[...]) - edges

        ev = jax.nn.softplus(jnp.dot(edges, wv_ref[...],
                                     preferred_element_type=f32))
        evc = ev * c_ref[...]
        wn_u = jnp.dot(rt_ref[...], evc, preferred_element_type=f32)
        deg = jnp.sum(wn_u, axis=1, keepdims=True)
        wn = jnp.where(deg > 0.0, 1.0 / deg, 0.0) * wn_u
        msg_ref[0, g * N:(g + 1) * N] = jnp.dot(
            wn, x, preferred_element_type=f32)


def _node_kernel(big_ref, msg_ref, z0_ref, w2_ref, w3_ref, out_ref):
    f32 = jnp.float32
    out_ref[0] = jnp.tanh(
        jnp.dot(msg_ref[0], w2_ref[...], preferred_element_type=f32)
        + jnp.dot(z0_ref[0], w3_ref[...], preferred_element_type=f32))


def _run(z, treat_sel, node_z0, WeA, WeB, WeC, WeD, W1A, W1B, be2, wv2,
         P, Rt, C, W2, W3):
    znode3 = z[:K_N].reshape(K // G, GN, D)
    treat3 = treat_sel.reshape(K // G, GN, TDIM)
    z3 = z.reshape(K // G + 1, GNN, D)
    big, msg = pl.pallas_call(
        _edge_kernel,
        grid=(K // G,),
        in_specs=[
            pl.BlockSpec((1, GN, D), lambda k: (k, 0, 0)),
            pl.BlockSpec((1, GNN, D), lambda k: (k + 1, 0, 0)),
            pl.BlockSpec((1, GN, TDIM), lambda k: (k, 0, 0)),
            pl.BlockSpec((D, D), lambda k: (0, 0)),
            pl.BlockSpec((TDIM, D), lambda k: (0, 0)),
            pl.BlockSpec((D, D), lambda k: (0, 0)),
            pl.BlockSpec((TDIM, D), lambda k: (0, 0)),
            pl.BlockSpec((D, D), lambda k: (0, 0)),
            pl.BlockSpec((TDIM, D), lambda k: (0, 0)),
            pl.BlockSpec((1, D), lambda k: (0, 0)),
            pl.BlockSpec((D, 1), lambda k: (0, 0)),
            pl.BlockSpec((NN, 2 * N), lambda k: (0, 0)),
            pl.BlockSpec((N, NN), lambda k: (0, 0)),
            pl.BlockSpec((NN, N), lambda k: (0, 0)),
        ],
        out_specs=[
            pl.BlockSpec((1, GNN, D), lambda k: (k + 1, 0, 0)),
            pl.BlockSpec((1, GN, D), lambda k: (k, 0, 0)),
        ],
        out_shape=[
            jax.ShapeDtypeStruct((K // G + 1, GNN, D), jnp.float32),
            jax.ShapeDtypeStruct((K // G, GN, D), jnp.float32),
        ],
        compiler_params=pltpu.CompilerParams(
            dimension_semantics=("parallel",)),
    )(znode3, z3, treat3, WeA, WeB, WeC, WeD, W1A, W1B, be2, wv2, P, Rt, C)
    msg2 = msg.reshape(2, NN, D)
    z02 = node_z0.reshape(2, NN, D)
    big2 = big.reshape(K + 2, NN, D)

    grad = pl.pallas_call(
        _node_kernel,
        grid=(2,),
        in_specs=[
            pl.BlockSpec((1, NN, D), lambda i: (i, 0, 0)),
            pl.BlockSpec((1, NN, D), lambda i: (i, 0, 0)),
            pl.BlockSpec((1, NN, D), lambda i: (i, 0, 0)),
            pl.BlockSpec((D, D), lambda i: (0, 0)),
            pl.BlockSpec((D, D), lambda i: (0, 0)),
        ],
        out_specs=pl.BlockSpec((1, NN, D), lambda i: (i, 0, 0)),
        out_shape=jax.ShapeDtypeStruct((K + 2, NN, D), jnp.float32),
        input_output_aliases={0: 0},
        compiler_params=pltpu.CompilerParams(
            dimension_semantics=("parallel",)),
    )(big2, msg2, z02, W2, W3)
    return grad.reshape(K_N + E, D)


def kernel(t_local, z, time_steps_to_predict, t_treatments, node_z0,
           We, be, w_v, W1, W2, W3, row, col):
    cin = D + TDIM
    t_index = jnp.maximum(
        jnp.sum(t_local[0] >= time_steps_to_predict) - 1, 0)
    treat_sel = jax.lax.dynamic_index_in_dim(
        t_treatments, t_index, axis=1, keepdims=False)

    WeA = We[:D]
    WeB = We[D:cin]
    WeC = We[cin:cin + D]
    WeD = We[cin + D:]
    W1A = W1[:D]
    W1B = W1[D:]
    be2 = be[None, :]
    wv2 = w_v[:, None]

    e_idx = np.arange(NN)
    P_np = np.zeros((NN, 2 * N), dtype=np.float32)
    P_np[e_idx, e_idx // N] = 1.0
    P_np[e_idx, N + e_idx % N] = 1.0
    Rt_np = np.zeros((N, NN), dtype=np.float32)
    Rt_np[e_idx // N, e_idx] = 1.0
    C_np = np.zeros((NN, N), dtype=np.float32)
    C_np[e_idx, e_idx % N] = 1.0
    P = jnp.asarray(P_np)
    Rt = jnp.asarray(Rt_np)
    C = jnp.asarray(C_np)

    return _run(z, treat_sel, node_z0, WeA, WeB, WeC, WeD, W1A, W1B,
                be2, wv2, P, Rt, C, W2, W3)
